# Initial kernel scaffold; baseline (speedup 1.0000x reference)
import jax
import jax.numpy as jnp
from jax.experimental import pallas as pl


def kernel(x, edge_index, W_proj, scoring_src, scoring_trg, bias):
    raise NotImplementedError("write your pallas kernel here")


# speedup vs baseline: 35.4743x; 35.4743x over previous
import functools

import jax
import jax.numpy as jnp
from jax import lax
from jax.experimental import pallas as pl
from jax.experimental.pallas import tpu as pltpu
from jax.experimental.pallas import tpu_sc as plsc

H = 8
F = 16
D = H * F
NC = 2
NS = 16
CH = 128


def _proj_body(x_ref, w_ref, asrc_ref, atrg_ref, xp_ref, ss_ref, st_ref):
    xp = jnp.dot(x_ref[...], w_ref[...], preferred_element_type=jnp.float32)
    xp_ref[...] = xp
    ss_ref[...] = jnp.dot(xp, asrc_ref[...], preferred_element_type=jnp.float32)
    st_ref[...] = jnp.dot(xp, atrg_ref[...], preferred_element_type=jnp.float32)


def _project(x_pad, w, a_src, a_trg, np_, blk):
    grid = np_ // blk
    return pl.pallas_call(
        _proj_body,
        grid=(grid,),
        in_specs=[
            pl.BlockSpec((blk, D), lambda i: (i, 0)),
            pl.BlockSpec((D, D), lambda i: (0, 0)),
            pl.BlockSpec((D, F), lambda i: (0, 0)),
            pl.BlockSpec((D, F), lambda i: (0, 0)),
        ],
        out_specs=[
            pl.BlockSpec((blk, D), lambda i: (i, 0)),
            pl.BlockSpec((blk, F), lambda i: (i, 0)),
            pl.BlockSpec((blk, F), lambda i: (i, 0)),
        ],
        out_shape=[
            jax.ShapeDtypeStruct((np_, D), jnp.float32),
            jax.ShapeDtypeStruct((np_, F), jnp.float32),
            jax.ShapeDtypeStruct((np_, F), jnp.float32),
        ],
    )(x_pad, w, a_src, a_trg)


def _sc_body(nch, nh, rows_pt, ss_hbm, st_hbm, xp_hbm, src_hbm, trg_hbm,
             bias_hbm, out_hbm,
             src_v, trg_v, adj_v, ssb, stb, exb, xpb, zb, zb2, bias_v, sem,
             out_sh, den_sh):
    cid = lax.axis_index("c")
    sid = lax.axis_index("s")
    ept = nch * CH
    lo = cid * nh

    pltpu.sync_copy(bias_hbm, bias_v)

    def zrow(r, _):
        for k in range(D // 16):
            zb[r, pl.ds(16 * k, 16)] = jnp.zeros((16,), jnp.float32)
        zb2[r, :] = jnp.zeros((16,), jnp.float32)
        return 0
    lax.fori_loop(0, rows_pt, zrow, 0)
    pltpu.sync_copy(zb, out_sh.at[pl.ds(sid * rows_pt, rows_pt)])
    pltpu.sync_copy(zb2, den_sh.at[pl.ds(sid * rows_pt, rows_pt)])
    plsc.subcore_barrier()

    def chunk(i, _):
        base = sid * ept + i * CH
        pltpu.sync_copy(src_hbm.at[pl.ds(base, CH)], src_v)
        pltpu.sync_copy(trg_hbm.at[pl.ds(base, CH)], trg_v)
        c_ss = pltpu.async_copy(ss_hbm.at[src_v], ssb, sem)
        c_st = pltpu.async_copy(st_hbm.at[trg_v], stb, sem)
        c_xp = pltpu.async_copy(xp_hbm.at[src_v], xpb, sem)
        c_ss.wait()
        c_st.wait()
        c_xp.wait()

        def edge(e, _):
            s = ssb[e, :] + stb[e, :]
            ex = jnp.exp(jnp.maximum(s, 0.2 * s))
            exb[e, :] = ex
            for h in range(H):
                sc = ex[h]
                xpb[e, pl.ds(16 * h, 16)] = xpb[e, pl.ds(16 * h, 16)] * sc
            return 0
        lax.fori_loop(0, CH, edge, 0)

        for v in range(CH // 16):
            rel = trg_v[pl.ds(16 * v, 16)] - lo
            keep = (rel >= 0) & (rel < nh)
            adj_v[pl.ds(16 * v, 16)] = jnp.where(keep, rel, nh)

        pltpu.sync_copy(exb, den_sh.at[adj_v], add=True)
        pltpu.sync_copy(xpb, out_sh.at[adj_v], add=True)
        return 0
    lax.fori_loop(0, nch, chunk, 0)
    plsc.subcore_barrier()

    r0 = sid * rows_pt
    pltpu.sync_copy(out_sh.at[pl.ds(r0, rows_pt)], zb)
    pltpu.sync_copy(den_sh.at[pl.ds(r0, rows_pt)], zb2)

    def frow(r, _):
        dv = zb2[r, :]
        for h in range(H):
            dh = dv[h] + 1e-16
            val = zb[r, pl.ds(16 * h, 16)] / dh + bias_v[pl.ds(16 * h, 16)]
            zb[r, pl.ds(16 * h, 16)] = jnp.where(
                val > 0, val, jnp.exp(val) - 1.0)
        return 0
    lax.fori_loop(0, rows_pt, frow, 0)
    pltpu.sync_copy(zb, out_hbm.at[pl.ds(lo + r0, rows_pt)])


def _sc_edge_pass(ss, st, xp, src, trg, bias, np_, nch):
    nh = np_ // NC
    rows_pt = nh // NS
    mesh = plsc.VectorSubcoreMesh(core_axis_name="c", subcore_axis_name="s")
    fn = pl.kernel(
        functools.partial(_sc_body, nch, nh, rows_pt),
        out_type=jax.ShapeDtypeStruct((np_, D), jnp.float32),
        mesh=mesh,
        compiler_params=pltpu.CompilerParams(use_tc_tiling_on_sc=False),
        scratch_types=[
            pltpu.VMEM((CH,), jnp.int32),
            pltpu.VMEM((CH,), jnp.int32),
            pltpu.VMEM((CH,), jnp.int32),
            pltpu.VMEM((CH, F), jnp.float32),
            pltpu.VMEM((CH, F), jnp.float32),
            pltpu.VMEM((CH, F), jnp.float32),
            pltpu.VMEM((CH, D), jnp.float32),
            pltpu.VMEM((nh // NS, D), jnp.float32),
            pltpu.VMEM((nh // NS, F), jnp.float32),
            pltpu.VMEM((D,), jnp.float32),
            pltpu.SemaphoreType.DMA,
            pltpu.VMEM_SHARED((nh + 16, D), jnp.float32),
            pltpu.VMEM_SHARED((nh + 16, F), jnp.float32),
        ],
    )
    return fn(ss, st, xp, src, trg, bias)


def kernel(x, edge_index, W_proj, scoring_src, scoring_trg, bias):
    n, d_in = x.shape
    e = edge_index.shape[1]
    assert d_in == D and W_proj.shape == (d_in, D)

    blk = 512
    np_ = ((n + 1024 - 1) // 1024) * 1024
    ept = ((e + NS * CH - 1) // (NS * CH)) * CH
    nch = ept // CH
    e_pad = ept * NS

    hsel = (jnp.arange(D)[:, None] // F == jnp.arange(F)[None, :])
    a_src = jnp.where(hsel, scoring_src.reshape(-1)[:, None], 0.0).astype(jnp.float32)
    a_trg = jnp.where(hsel, scoring_trg.reshape(-1)[:, None], 0.0).astype(jnp.float32)

    x_pad = jnp.concatenate(
        [x, jnp.zeros((np_ - n, d_in), jnp.float32)], axis=0)
    pad_idx = jnp.full((e_pad - e,), n, jnp.int32)
    src = jnp.concatenate([edge_index[0], pad_idx])
    trg = jnp.concatenate([edge_index[1], pad_idx])

    xp, ss, st = _project(x_pad, W_proj, a_src, a_trg, np_, blk)
    out = _sc_edge_pass(ss, st, xp, src, trg,
                        bias.astype(jnp.float32), np_, nch)
    return out[:n]


# speedup vs baseline: 44.6146x; 1.2577x over previous
import functools

import jax
import jax.numpy as jnp
from jax import lax
from jax.experimental import pallas as pl
from jax.experimental.pallas import tpu as pltpu
from jax.experimental.pallas import tpu_sc as plsc

H = 8
F = 16
D = H * F
NC = 2
NS = 16
CH = 128


def _proj_body(x_ref, w_ref, asrc_ref, atrg_ref, xp_ref, ss_ref, st_ref):
    xp = jnp.dot(x_ref[...], w_ref[...], preferred_element_type=jnp.float32)
    xp_ref[...] = xp
    ss_ref[...] = jnp.dot(xp, asrc_ref[...], preferred_element_type=jnp.float32)
    st_ref[...] = jnp.dot(xp, atrg_ref[...], preferred_element_type=jnp.float32)


def _project(x_pad, w, a_src, a_trg, np_, blk):
    grid = np_ // blk
    return pl.pallas_call(
        _proj_body,
        grid=(grid,),
        in_specs=[
            pl.BlockSpec((blk, D), lambda i: (i, 0)),
            pl.BlockSpec((D, D), lambda i: (0, 0)),
            pl.BlockSpec((D, F), lambda i: (0, 0)),
            pl.BlockSpec((D, F), lambda i: (0, 0)),
        ],
        out_specs=[
            pl.BlockSpec((blk, D), lambda i: (i, 0)),
            pl.BlockSpec((blk, F), lambda i: (i, 0)),
            pl.BlockSpec((blk, F), lambda i: (i, 0)),
        ],
        out_shape=[
            jax.ShapeDtypeStruct((np_, D), jnp.float32),
            jax.ShapeDtypeStruct((np_, F), jnp.float32),
            jax.ShapeDtypeStruct((np_, F), jnp.float32),
        ],
    )(x_pad, w, a_src, a_trg)


def _sc_body(nch, nh, rows_pt, ss_hbm, st_hbm, xp_hbm, src_hbm, trg_hbm,
             bias_hbm, out_hbm,
             src0, src1, trg0, trg1, adj0, adj1, ssb0, ssb1, stb0, stb1,
             exb0, exb1, xpb0, xpb1, zb, zb2, bias_v,
             semi0, semi1, semg0, semg1, sems0, sems1,
             out_sh, den_sh):
    cid = lax.axis_index("c")
    sid = lax.axis_index("s")
    ept = nch * CH
    lo = cid * nh

    srcv = (src0, src1)
    trgv = (trg0, trg1)
    adjv = (adj0, adj1)
    ssb = (ssb0, ssb1)
    stb = (stb0, stb1)
    exb = (exb0, exb1)
    xpb = (xpb0, xpb1)
    semi = (semi0, semi1)
    semg = (semg0, semg1)
    sems = (sems0, sems1)

    pltpu.sync_copy(bias_hbm, bias_v)

    hrows = rows_pt // 2
    def zrow(r, _):
        for k in range(D // 16):
            zb[r, pl.ds(16 * k, 16)] = jnp.zeros((16,), jnp.float32)
        zb2[r, :] = jnp.zeros((16,), jnp.float32)
        return 0
    lax.fori_loop(0, hrows, zrow, 0)
    for half in (0, 1):
        pltpu.sync_copy(zb, out_sh.at[pl.ds(sid * rows_pt + half * hrows, hrows)])
        pltpu.sync_copy(zb2, den_sh.at[pl.ds(sid * rows_pt + half * hrows, hrows)])
    plsc.subcore_barrier()

    def issue_idx(i, b):
        base = sid * ept + i * CH
        pltpu.async_copy(src_hbm.at[pl.ds(base, CH)], srcv[b], semi[b])
        pltpu.async_copy(trg_hbm.at[pl.ds(base, CH)], trgv[b], semi[b])

    def wait_idx(b):
        pltpu.make_async_copy(src_hbm.at[pl.ds(0, CH)], srcv[b], semi[b]).wait()
        pltpu.make_async_copy(trg_hbm.at[pl.ds(0, CH)], trgv[b], semi[b]).wait()

    def issue_gathers(b):
        pltpu.async_copy(ss_hbm.at[srcv[b]], ssb[b], semg[b])
        pltpu.async_copy(st_hbm.at[trgv[b]], stb[b], semg[b])
        pltpu.async_copy(xp_hbm.at[srcv[b]], xpb[b], semg[b])

    def wait_gathers(b):
        pltpu.make_async_copy(ss_hbm.at[srcv[b]], ssb[b], semg[b]).wait()
        pltpu.make_async_copy(st_hbm.at[trgv[b]], stb[b], semg[b]).wait()
        pltpu.make_async_copy(xp_hbm.at[srcv[b]], xpb[b], semg[b]).wait()

    def issue_scatters(b):
        pltpu.async_copy(exb[b], den_sh.at[adjv[b]], sems[b], add=True)
        pltpu.async_copy(xpb[b], out_sh.at[adjv[b]], sems[b], add=True)

    def wait_scatters(b):
        pltpu.make_async_copy(exb[b], den_sh.at[adjv[b]], sems[b]).wait()
        pltpu.make_async_copy(xpb[b], out_sh.at[adjv[b]], sems[b]).wait()

    def compute(b):
        for v in range(CH // 16):
            rel = trgv[b][pl.ds(16 * v, 16)] - lo
            keep = (rel >= 0) & (rel < nh)
            adjv[b][pl.ds(16 * v, 16)] = jnp.where(keep, rel, nh)

        def edge(e, _):
            s = ssb[b][e, :] + stb[b][e, :]
            ex = jnp.exp(jnp.maximum(s, 0.2 * s))
            exb[b][e, :] = ex
            for h in range(H):
                sc = ex[h]
                xpb[b][e, pl.ds(16 * h, 16)] = (
                    xpb[b][e, pl.ds(16 * h, 16)] * sc)
            return 0
        lax.fori_loop(0, CH, edge, 0, unroll=2)

    issue_idx(0, 0)
    issue_idx(1, 1)
    wait_idx(0)
    issue_gathers(0)

    def pair(k, _):
        for b in (0, 1):
            i = 2 * k + b
            wait_gathers(b)
            wait_idx(1 - b)

            @pl.when(i > 0)
            def _():
                wait_scatters(1 - b)

            issue_gathers(1 - b)
            compute(b)
            issue_scatters(b)
            issue_idx(i + 2, b)
        return 0
    lax.fori_loop(0, nch // 2, pair, 0)

    wait_scatters(1)
    wait_gathers(0)
    wait_idx(1)
    plsc.subcore_barrier()

    def frow(r, _):
        dv = zb2[r, :]
        for h in range(H):
            dh = dv[h] + 1e-16
            val = zb[r, pl.ds(16 * h, 16)] / dh + bias_v[pl.ds(16 * h, 16)]
            zb[r, pl.ds(16 * h, 16)] = jnp.where(
                val > 0, val, jnp.exp(val) - 1.0)
        return 0
    r0 = sid * rows_pt
    for half in (0, 1):
        pltpu.sync_copy(out_sh.at[pl.ds(r0 + half * hrows, hrows)], zb)
        pltpu.sync_copy(den_sh.at[pl.ds(r0 + half * hrows, hrows)], zb2)
        lax.fori_loop(0, hrows, frow, 0)
        pltpu.sync_copy(zb, out_hbm.at[pl.ds(lo + r0 + half * hrows, hrows)])


def _sc_edge_pass(ss, st, xp, src, trg, bias, np_, nch):
    nh = np_ // NC
    rows_pt = nh // NS
    mesh = plsc.VectorSubcoreMesh(core_axis_name="c", subcore_axis_name="s")
    fn = pl.kernel(
        functools.partial(_sc_body, nch, nh, rows_pt),
        out_type=jax.ShapeDtypeStruct((np_, D), jnp.float32),
        mesh=mesh,
        compiler_params=pltpu.CompilerParams(use_tc_tiling_on_sc=False),
        scratch_types=(
            [pltpu.VMEM((CH,), jnp.int32)] * 6
            + [pltpu.VMEM((CH, F), jnp.float32)] * 6
            + [pltpu.VMEM((CH, D), jnp.float32)] * 2
            + [
                pltpu.VMEM((nh // NS // 2, D), jnp.float32),
                pltpu.VMEM((nh // NS // 2, F), jnp.float32),
                pltpu.VMEM((D,), jnp.float32),
            ]
            + [pltpu.SemaphoreType.DMA] * 6
            + [
                pltpu.VMEM_SHARED((nh + 8, D), jnp.float32),
                pltpu.VMEM_SHARED((nh + 8, F), jnp.float32),
            ]
        ),
    )
    return fn(ss, st, xp, src, trg, bias)


def kernel(x, edge_index, W_proj, scoring_src, scoring_trg, bias):
    n, d_in = x.shape
    e = edge_index.shape[1]
    assert d_in == D and W_proj.shape == (d_in, D)

    blk = 512
    np_ = ((n + 1024 - 1) // 1024) * 1024
    nch = -(-e // (NS * CH))
    nch = nch + (nch % 2)
    ept = nch * CH
    e_pad = ept * NS + 2 * CH

    hsel = (jnp.arange(D)[:, None] // F == jnp.arange(F)[None, :])
    a_src = jnp.where(hsel, scoring_src.reshape(-1)[:, None], 0.0).astype(jnp.float32)
    a_trg = jnp.where(hsel, scoring_trg.reshape(-1)[:, None], 0.0).astype(jnp.float32)

    x_pad = jnp.concatenate(
        [x, jnp.zeros((np_ - n, d_in), jnp.float32)], axis=0)
    pad_idx = jnp.full((e_pad - e,), n, jnp.int32)
    src = jnp.concatenate([edge_index[0], pad_idx])
    trg = jnp.concatenate([edge_index[1], pad_idx])

    xp, ss, st = _project(x_pad, W_proj, a_src, a_trg, np_, blk)
    out = _sc_edge_pass(ss, st, xp, src, trg,
                        bias.astype(jnp.float32), np_, nch)
    return out[:n]
